# baseline (speedup 1.0000x reference)
import jax
import jax.numpy as jnp
from jax.experimental import pallas as pl
from jax.experimental.pallas import tpu as pltpu

_B = 4096
_T = 50
_D_IN = 64
_D_MEM = 64
_NSPK = 10
_BLK = 1024


def _body(x_ref, sp_ref, w_ref, out_ref, mem_ref):
    mem_ref[...] = jnp.zeros_like(mem_ref)
    w = w_ref[...]
    d = _D_MEM
    pad = jnp.where(
        jax.lax.broadcasted_iota(jnp.int32, (8, _BLK), 0) == 0, 1.0, 0.0)
    for t in range(_T):
        xt = x_ref[t * _D_IN:(t + 1) * _D_IN, :]
        srow = sp_ref[t:t + 1, :]
        h = jnp.zeros((d, _BLK), jnp.float32)
        masks = []
        for s in range(_NSPK):
            m = srow == s
            masks.append(m)
            h = jnp.where(m, mem_ref[s], h)
        hx = jnp.concatenate([xt, h, pad], axis=0)
        g = jax.lax.dot_general(
            w, hx, (((1,), (0,)), ((), ())),
            preferred_element_type=jnp.float32)
        r = jax.nn.sigmoid(g[0:d] + g[3 * d:4 * d])
        z = jax.nn.sigmoid(g[d:2 * d] + g[4 * d:5 * d])
        n = jnp.tanh(g[2 * d:3 * d] + r * g[5 * d:6 * d])
        h_new = n + z * (h - n)
        for s in range(_NSPK):
            pltpu.store(
                mem_ref.at[s], h_new,
                mask=jnp.broadcast_to(masks[s], (d, _BLK)))
        out_ref[t * d:(t + 1) * d, :] = h_new


@jax.jit
def kernel(x_in, speakers, W_ih, W_hh, b_ih, b_hh):
    d = _D_MEM
    w = jnp.zeros((6 * d, _D_IN + d + 8), jnp.float32)
    w = w.at[:3 * d, :_D_IN].set(W_ih)
    w = w.at[3 * d:, _D_IN:_D_IN + d].set(W_hh)
    w = w.at[:, _D_IN + d].set(jnp.concatenate([b_ih, b_hh]))
    xT = x_in.reshape(_B, _T * _D_IN).T
    spT = speakers.astype(jnp.int32).T

    grid = (_B // _BLK,)
    out = pl.pallas_call(
        _body,
        grid=grid,
        in_specs=[
            pl.BlockSpec((_T * _D_IN, _BLK), lambda i: (0, i)),
            pl.BlockSpec((_T, _BLK), lambda i: (0, i)),
            pl.BlockSpec((6 * d, _D_IN + d + 8), lambda i: (0, 0)),
        ],
        out_specs=pl.BlockSpec((_T * d, _BLK), lambda i: (0, i)),
        out_shape=jax.ShapeDtypeStruct((_T * d, _B), jnp.float32),
        compiler_params=pltpu.CompilerParams(
            dimension_semantics=("parallel",),
            vmem_limit_bytes=115 * 1024 * 1024),
        scratch_shapes=[pltpu.VMEM((_NSPK, d, _BLK), jnp.float32)],
    )(xT, spT, w)
    return out.T.reshape(_B, _T, d)


# speedup vs baseline: 1.3973x; 1.3973x over previous
import jax
import jax.numpy as jnp
from jax.experimental import pallas as pl
from jax.experimental.pallas import tpu as pltpu

_B = 4096
_T = 50
_D_IN = 64
_D_MEM = 64
_NSPK = 10
_BLK = 1024


def _body(x_ref, sp_ref, w_ref, out_ref, mem_ref):
    w = w_ref[...]
    d = _D_MEM
    pad = jnp.where(
        jax.lax.broadcasted_iota(jnp.int32, (8, _BLK), 0) == 0, 1.0, 0.0)
    h = jnp.zeros((d, _BLK), jnp.float32)
    masks = [sp_ref[0:1, :] == s for s in range(_NSPK)]
    for t in range(_T):
        xt = x_ref[t * _D_IN:(t + 1) * _D_IN, :]
        hx = jnp.concatenate([xt, h, pad], axis=0)
        g = jax.lax.dot_general(
            w, hx, (((1,), (0,)), ((), ())),
            preferred_element_type=jnp.float32)
        r = jax.nn.sigmoid(g[0:d] + g[3 * d:4 * d])
        z = jax.nn.sigmoid(g[d:2 * d] + g[4 * d:5 * d])
        n = jnp.tanh(g[2 * d:3 * d] + r * g[5 * d:6 * d])
        h_new = n + z * (h - n)
        out_ref[t * d:(t + 1) * d, :] = h_new
        if t < _T - 1:
            srow_n = sp_ref[t + 1:t + 2, :]
            nmasks = []
            h = jnp.zeros((d, _BLK), jnp.float32)
            for s in range(_NSPK):
                old = mem_ref[s] if t > 0 else jnp.zeros((d, _BLK),
                                                         jnp.float32)
                new_s = jnp.where(masks[s], h_new, old)
                mem_ref[s] = new_s
                mn = srow_n == s
                nmasks.append(mn)
                h = jnp.where(mn, new_s, h)
            masks = nmasks


@jax.jit
def kernel(x_in, speakers, W_ih, W_hh, b_ih, b_hh):
    d = _D_MEM
    w = jnp.zeros((6 * d, _D_IN + d + 8), jnp.float32)
    w = w.at[:3 * d, :_D_IN].set(W_ih)
    w = w.at[3 * d:, _D_IN:_D_IN + d].set(W_hh)
    w = w.at[:, _D_IN + d].set(jnp.concatenate([b_ih, b_hh]))
    xT = x_in.reshape(_B, _T * _D_IN).T
    spT = speakers.astype(jnp.int32).T

    grid = (_B // _BLK,)
    out = pl.pallas_call(
        _body,
        grid=grid,
        in_specs=[
            pl.BlockSpec((_T * _D_IN, _BLK), lambda i: (0, i)),
            pl.BlockSpec((_T, _BLK), lambda i: (0, i)),
            pl.BlockSpec((6 * d, _D_IN + d + 8), lambda i: (0, 0)),
        ],
        out_specs=pl.BlockSpec((_T * d, _BLK), lambda i: (0, i)),
        out_shape=jax.ShapeDtypeStruct((_T * d, _B), jnp.float32),
        compiler_params=pltpu.CompilerParams(
            dimension_semantics=("parallel",),
            vmem_limit_bytes=115 * 1024 * 1024),
        scratch_shapes=[pltpu.VMEM((_NSPK, d, _BLK), jnp.float32)],
    )(xT, spT, w)
    return out.T.reshape(_B, _T, d)


# speedup vs baseline: 1.4383x; 1.0294x over previous
import jax
import jax.numpy as jnp
from jax.experimental import pallas as pl
from jax.experimental.pallas import tpu as pltpu

_B = 4096
_T = 50
_D_IN = 64
_D_MEM = 64
_NSPK = 10
_BLK = 1024


def _body(x_ref, sp_ref, w_ref, out_ref, mem_ref):
    w = w_ref[...]
    d = _D_MEM
    pad = jnp.where(
        jax.lax.broadcasted_iota(jnp.int32, (8, _BLK), 0) == 0, 1.0, 0.0)
    h = jnp.zeros((d, _BLK), jnp.float32)
    masks = [sp_ref[0:1, :] == s for s in range(_NSPK)]
    for t in range(_T):
        xt = x_ref[t * _D_IN:(t + 1) * _D_IN, :]
        hx = jnp.concatenate([xt, h, pad], axis=0)
        g = jax.lax.dot_general(
            w, hx, (((1,), (0,)), ((), ())),
            preferred_element_type=jnp.float32)
        r = jax.nn.sigmoid(g[0:d] + g[3 * d:4 * d])
        z = jax.nn.sigmoid(g[d:2 * d] + g[4 * d:5 * d])
        n = jnp.tanh(g[2 * d:3 * d] + r * g[5 * d:6 * d])
        h_new = n + z * (h - n)
        out_ref[t * d:(t + 1) * d, :] = h_new
        if t < _T - 1:
            srow_n = sp_ref[t + 1:t + 2, :]
            nmasks = []
            parts = []
            for s in range(_NSPK):
                old = mem_ref[s] if t > 0 else jnp.zeros((d, _BLK),
                                                         jnp.float32)
                new_s = jnp.where(masks[s], h_new, old)
                mem_ref[s] = new_s
                mn = srow_n == s
                nmasks.append(mn)
                parts.append(new_s * mn.astype(jnp.float32))
            masks = nmasks
            while len(parts) > 1:
                parts = [a + b for a, b in zip(parts[::2], parts[1::2])] + (
                    [parts[-1]] if len(parts) % 2 else [])
            h = parts[0]


@jax.jit
def kernel(x_in, speakers, W_ih, W_hh, b_ih, b_hh):
    d = _D_MEM
    w = jnp.zeros((6 * d, _D_IN + d + 8), jnp.float32)
    w = w.at[:3 * d, :_D_IN].set(W_ih)
    w = w.at[3 * d:, _D_IN:_D_IN + d].set(W_hh)
    w = w.at[:, _D_IN + d].set(jnp.concatenate([b_ih, b_hh]))
    xT = x_in.reshape(_B, _T * _D_IN).T
    spT = speakers.astype(jnp.int32).T

    grid = (_B // _BLK,)
    out = pl.pallas_call(
        _body,
        grid=grid,
        in_specs=[
            pl.BlockSpec((_T * _D_IN, _BLK), lambda i: (0, i)),
            pl.BlockSpec((_T, _BLK), lambda i: (0, i)),
            pl.BlockSpec((6 * d, _D_IN + d + 8), lambda i: (0, 0)),
        ],
        out_specs=pl.BlockSpec((_T * d, _BLK), lambda i: (0, i)),
        out_shape=jax.ShapeDtypeStruct((_T * d, _B), jnp.float32),
        compiler_params=pltpu.CompilerParams(
            dimension_semantics=("parallel",),
            vmem_limit_bytes=115 * 1024 * 1024),
        scratch_shapes=[pltpu.VMEM((_NSPK, d, _BLK), jnp.float32)],
    )(xT, spT, w)
    return out.T.reshape(_B, _T, d)
